# Initial kernel scaffold; baseline (speedup 1.0000x reference)
import jax
import jax.numpy as jnp
from jax.experimental import pallas as pl


def kernel(x, W_cq, s_q, W_dq_nope, W_dq_rope, W_ckv, s_kv, W_dk_nope, W_dv, W_krope, W_imp, b_imp, W_selk, W_selv, W_wink, W_winv, W_gate, b_gate, W_proj):
    raise NotImplementedError("write your pallas kernel here")


# speedup vs baseline: 1.0549x; 1.0549x over previous
import functools

import jax
import jax.numpy as jnp
from jax.experimental import pallas as pl
from jax.experimental.pallas import tpu as pltpu

_VMEM_BIG = pltpu.CompilerParams(vmem_limit_bytes=120 * 1024 * 1024)

T = 2048
C = 1024
N_HEAD = 16
D_PAD = 128
HP = N_HEAD * D_PAD
K_KEEP = 512
QB = 256
ROPE_HALF = 32
NOPE = 32
SCALE = 1.0 / (96.0 ** 0.5)
NEG = -1e9


def _pack_qk_weights(w_nope, w_rope):
    d = w_nope.shape[0]
    nope = w_nope.reshape(d, N_HEAD, NOPE)
    rope = w_rope.reshape(d, N_HEAD, 2 * ROPE_HALF)
    real = rope[:, :, :ROPE_HALF]
    imag = rope[:, :, ROPE_HALF:]
    z = jnp.zeros_like(nope)
    a = jnp.concatenate([nope, real, imag, z], axis=-1).reshape(d, HP)
    b = jnp.concatenate([z, imag, real, z], axis=-1).reshape(d, HP)
    return a, b


def _pack_qk_weights_fused(w):
    d = w.shape[0]
    w3 = w.reshape(d, N_HEAD, NOPE + 2 * ROPE_HALF)
    return _pack_qk_weights(
        w3[:, :, :NOPE].reshape(d, N_HEAD * NOPE),
        w3[:, :, NOPE:].reshape(d, N_HEAD * 2 * ROPE_HALF),
    )


def _pack_v_weights(w):
    d = w.shape[0]
    w3 = w.reshape(d, N_HEAD, 96)
    z = jnp.zeros((d, N_HEAD, D_PAD - 96), w.dtype)
    return jnp.concatenate([w3, z], axis=-1).reshape(d, HP)


def _rope_tables(n):
    freqs = 1.0 / 10000.0 ** (jnp.arange(0, 64, 2, dtype=jnp.float32) / 64.0)
    t = jnp.arange(n, dtype=jnp.float32)
    ang = jnp.outer(t, freqs)
    cos, sin = jnp.cos(ang), jnp.sin(ang)
    one = jnp.ones_like(cos)
    zero = jnp.zeros_like(cos)
    cos_blk = jnp.concatenate([one, cos, cos, zero], axis=-1)
    sin_blk = jnp.concatenate([zero, -sin, sin, zero], axis=-1)
    return jnp.tile(cos_blk, (1, N_HEAD)), jnp.tile(sin_blk, (1, N_HEAD))


def _proj_kernel(x_ref, wcq_ref, wqa_ref, wqb_ref, wckv_ref, wka_ref, wkb_ref,
                 wv_ref, wwa_ref, wwb_ref, wwv_ref, wimp_ref, wgate_ref,
                 cos_ref, sin_ref,
                 q_ref, k1_ref, v1_ref, kw_ref, vw_ref, imp_ref, gate_ref):
    xb = x_ref[...]
    cosb = cos_ref[...]
    sinb = sin_ref[...]

    def rms(v):
        return jax.lax.rsqrt(jnp.mean(v * v, axis=-1, keepdims=True) + 1e-6)

    cq = jnp.dot(xb, wcq_ref[...], preferred_element_type=jnp.float32)
    nq = cq * rms(cq)
    q_ref[...] = (jnp.dot(nq, wqa_ref[...], preferred_element_type=jnp.float32) * cosb
                  + jnp.dot(nq, wqb_ref[...], preferred_element_type=jnp.float32) * sinb)

    ckv = jnp.dot(xb, wckv_ref[...], preferred_element_type=jnp.float32)
    nkv = ckv * rms(ckv)
    k1_ref[...] = (jnp.dot(nkv, wka_ref[...], preferred_element_type=jnp.float32) * cosb
                   + jnp.dot(nkv, wkb_ref[...], preferred_element_type=jnp.float32) * sinb)
    v1_ref[...] = jnp.dot(nkv, wv_ref[...], preferred_element_type=jnp.float32)

    kw_ref[...] = (jnp.dot(xb, wwa_ref[...], preferred_element_type=jnp.float32) * cosb
                   + jnp.dot(xb, wwb_ref[...], preferred_element_type=jnp.float32) * sinb)
    vw_ref[...] = jnp.dot(xb, wwv_ref[...], preferred_element_type=jnp.float32)

    imp_ref[...] = jnp.dot(xb, wimp_ref[...], preferred_element_type=jnp.float32)
    gp = jnp.dot(xb, wgate_ref[...], preferred_element_type=jnp.float32)
    gate_ref[...] = jnp.sum(gp, axis=0, keepdims=True).reshape(1, 1, 128)


def _topk_kernel(icol_ref, irow_ref, x_ref, cnt_ref, selx_ref):
    fcol = icol_ref[:, :1]
    frow = irow_ref[:1, :]
    isub = jax.lax.broadcasted_iota(jnp.int32, (T, T), 0)
    jlane = jax.lax.broadcasted_iota(jnp.int32, (T, T), 1)
    beats = jnp.where(
        (frow > fcol) | ((frow == fcol) & (jlane < isub)), 1.0, 0.0)
    rank_col = jnp.sum(beats, axis=1, keepdims=True)
    rank_row = (T - 1.0) - jnp.sum(beats, axis=0, keepdims=True)
    sel_col = jnp.where(rank_col < K_KEEP, 1.0, 0.0)
    sel_row = jnp.where(rank_row < K_KEEP, 1.0, 0.0)

    m = jnp.where(isub < jlane, 1.0, 0.0)
    sel_col128 = jnp.broadcast_to(sel_col, (T, 128))
    after = jnp.dot(m, sel_col128, preferred_element_type=jnp.float32)
    cnt_ref[...] = K_KEEP - after

    sel_row8 = jnp.broadcast_to(sel_row, (8, T))
    order8 = jnp.dot(sel_row8, m, preferred_element_type=jnp.float32)
    riota = jax.lax.broadcasted_iota(jnp.int32, (K_KEEP, T), 0).astype(jnp.float32)
    onehot = jnp.where((order8[:1, :] == riota) & (sel_row[:1, :] > 0.5), 1.0, 0.0)
    selx_ref[...] = jnp.dot(onehot, x_ref[...], preferred_element_type=jnp.float32)


def _selproj_kernel(selx_ref, wsa_ref, wsb_ref, wsv_ref, cos_ref, sin_ref,
                    ks_ref, vs_ref):
    sx = selx_ref[...]
    ks_ref[...] = (jnp.dot(sx, wsa_ref[...], preferred_element_type=jnp.float32) * cos_ref[...]
                   + jnp.dot(sx, wsb_ref[...], preferred_element_type=jnp.float32) * sin_ref[...])
    vs_ref[...] = jnp.dot(sx, wsv_ref[...], preferred_element_type=jnp.float32)


def _attn_kernel(q_ref, k1_ref, v1_ref, kw_ref, vw_ref, ks_ref, vs_ref,
                 cnt_ref, bw_ref, o_ref):
    qb = pl.program_id(1)
    qv = q_ref[...]
    row = qb * QB + jax.lax.broadcasted_iota(jnp.int32, (QB, T), 0)
    col = jax.lax.broadcasted_iota(jnp.int32, (QB, T), 1)
    causal = col <= row

    dims = (((1,), (1,)), ((), ()))

    def soft_attend(k, v, mask):
        s = jax.lax.dot_general(qv, k, dims, preferred_element_type=jnp.float32) * SCALE
        s = jnp.where(mask, s, NEG)
        m = jnp.max(s, axis=-1, keepdims=True)
        p = jnp.exp(s - m)
        p = p / jnp.sum(p, axis=-1, keepdims=True)
        return jnp.dot(p, v, preferred_element_type=jnp.float32)

    o1 = soft_attend(k1_ref[...], v1_ref[...], causal)
    o3 = soft_attend(kw_ref[...], vw_ref[...], causal)

    cnt = cnt_ref[:, :1]
    kidx = jax.lax.broadcasted_iota(jnp.int32, (QB, K_KEEP), 1).astype(jnp.float32)
    o2 = soft_attend(ks_ref[...], vs_ref[...], kidx < cnt)

    w1 = bw_ref[:1, 0:128]
    w2 = bw_ref[:1, 128:256]
    w3 = bw_ref[:1, 256:384]
    o_ref[...] = o1 * w1 + o2 * w2 + o3 * w3


def _outproj_kernel(o_ref, wp_ref, out_ref):
    out_ref[...] = jnp.dot(o_ref[...], wp_ref[...],
                           preferred_element_type=jnp.float32)


@functools.partial(jax.jit, static_argnames=())
def kernel(x, W_cq, s_q, W_dq_nope, W_dq_rope, W_ckv, s_kv, W_dk_nope, W_dv,
           W_krope, W_imp, b_imp, W_selk, W_selv, W_wink, W_winv, W_gate,
           b_gate, W_proj):
    f32 = jnp.float32
    x2 = x.reshape(T, C).astype(f32)

    wqa, wqb = _pack_qk_weights(W_dq_nope, W_dq_rope)
    wqa, wqb = s_q[:, None] * wqa, s_q[:, None] * wqb
    wka, wkb = _pack_qk_weights(W_dk_nope, W_krope)
    wka, wkb = s_kv[:, None] * wka, s_kv[:, None] * wkb
    wv = s_kv[:, None] * _pack_v_weights(W_dv)
    wwa, wwb = _pack_qk_weights_fused(W_wink)
    wwv = _pack_v_weights(W_winv)
    wsa, wsb = _pack_qk_weights_fused(W_selk)
    wsv = _pack_v_weights(W_selv)
    wp = jnp.concatenate(
        [W_proj.reshape(N_HEAD, 96, C),
         jnp.zeros((N_HEAD, D_PAD - 96, C), f32)], axis=1).reshape(HP, C)
    wimp = jnp.concatenate([W_imp, jnp.zeros((C, 127), f32)], axis=-1)
    wgate = jnp.concatenate([W_gate, jnp.zeros((C, 125), f32)], axis=-1)
    cos_t, sin_t = _rope_tables(T)

    nblk = T // QB

    def full2(shape):
        return pl.BlockSpec(shape, lambda i: (0, 0))

    q, k1, v1, kw, vw, imp, gate_p = pl.pallas_call(
        _proj_kernel,
        compiler_params=_VMEM_BIG,
        grid=(nblk,),
        in_specs=[
            pl.BlockSpec((QB, C), lambda i: (i, 0)),
            full2((C, 96)), full2((96, HP)), full2((96, HP)),
            full2((C, 32)), full2((32, HP)), full2((32, HP)), full2((32, HP)),
            full2((C, HP)), full2((C, HP)), full2((C, HP)),
            full2((C, 128)), full2((C, 128)),
            pl.BlockSpec((QB, HP), lambda i: (i, 0)),
            pl.BlockSpec((QB, HP), lambda i: (i, 0)),
        ],
        out_specs=[
            pl.BlockSpec((QB, HP), lambda i: (i, 0)),
            pl.BlockSpec((QB, HP), lambda i: (i, 0)),
            pl.BlockSpec((QB, HP), lambda i: (i, 0)),
            pl.BlockSpec((QB, HP), lambda i: (i, 0)),
            pl.BlockSpec((QB, HP), lambda i: (i, 0)),
            pl.BlockSpec((QB, 128), lambda i: (i, 0)),
            pl.BlockSpec((1, 1, 128), lambda i: (i, 0, 0)),
        ],
        out_shape=[
            jax.ShapeDtypeStruct((T, HP), f32),
            jax.ShapeDtypeStruct((T, HP), f32),
            jax.ShapeDtypeStruct((T, HP), f32),
            jax.ShapeDtypeStruct((T, HP), f32),
            jax.ShapeDtypeStruct((T, HP), f32),
            jax.ShapeDtypeStruct((T, 128), f32),
            jax.ShapeDtypeStruct((nblk, 1, 128), f32),
        ],
    )(x2, W_cq, wqa, wqb, W_ckv, wka, wkb, wv, wwa, wwb, wwv, wimp, wgate,
      cos_t, sin_t)

    glog = gate_p.reshape(nblk, 128).sum(axis=0)[:3] / T + b_gate
    bw3 = jax.nn.softmax(glog)
    bw = jnp.broadcast_to(jnp.repeat(bw3, 128)[None, :], (8, 384))

    imp_vec = imp[:, 0] + b_imp[0]
    icol = jnp.broadcast_to(imp_vec[:, None], (T, 128))
    irow = jnp.broadcast_to(imp_vec[None, :], (8, T))

    cnt, selx = pl.pallas_call(
        _topk_kernel,
        compiler_params=_VMEM_BIG,
        grid=(1,),
        in_specs=[full2((T, 128)), full2((8, T)), full2((T, C))],
        out_specs=[full2((T, 128)), full2((K_KEEP, C))],
        out_shape=[
            jax.ShapeDtypeStruct((T, 128), f32),
            jax.ShapeDtypeStruct((K_KEEP, C), f32),
        ],
    )(icol, irow, x2)

    ks, vs = pl.pallas_call(
        _selproj_kernel,
        compiler_params=_VMEM_BIG,
        grid=(1,),
        in_specs=[full2((K_KEEP, C)), full2((C, HP)), full2((C, HP)),
                  full2((C, HP)), full2((K_KEEP, HP)), full2((K_KEEP, HP))],
        out_specs=[full2((K_KEEP, HP)), full2((K_KEEP, HP))],
        out_shape=[
            jax.ShapeDtypeStruct((K_KEEP, HP), f32),
            jax.ShapeDtypeStruct((K_KEEP, HP), f32),
        ],
    )(selx, wsa, wsb, wsv, cos_t[:K_KEEP], sin_t[:K_KEEP])

    o = pl.pallas_call(
        _attn_kernel,
        grid=(N_HEAD, nblk),
        in_specs=[
            pl.BlockSpec((QB, D_PAD), lambda h, i: (i, h)),
            pl.BlockSpec((T, D_PAD), lambda h, i: (0, h)),
            pl.BlockSpec((T, D_PAD), lambda h, i: (0, h)),
            pl.BlockSpec((T, D_PAD), lambda h, i: (0, h)),
            pl.BlockSpec((T, D_PAD), lambda h, i: (0, h)),
            pl.BlockSpec((K_KEEP, D_PAD), lambda h, i: (0, h)),
            pl.BlockSpec((K_KEEP, D_PAD), lambda h, i: (0, h)),
            pl.BlockSpec((QB, 128), lambda h, i: (i, 0)),
            pl.BlockSpec((8, 384), lambda h, i: (0, 0)),
        ],
        out_specs=pl.BlockSpec((QB, D_PAD), lambda h, i: (i, h)),
        out_shape=jax.ShapeDtypeStruct((T, HP), f32),
    )(q, k1, v1, kw, vw, ks, vs, cnt, bw)

    out = pl.pallas_call(
        _outproj_kernel,
        grid=(nblk,),
        in_specs=[pl.BlockSpec((QB, HP), lambda i: (i, 0)), full2((HP, C))],
        out_specs=pl.BlockSpec((QB, C), lambda i: (i, 0)),
        out_shape=jax.ShapeDtypeStruct((T, C), f32),
    )(o, wp)

    return out.reshape(1, T, C)


# speedup vs baseline: 1.1474x; 1.0877x over previous
import functools

import jax
import jax.numpy as jnp
from jax.experimental import pallas as pl
from jax.experimental.pallas import tpu as pltpu

_VMEM_BIG = pltpu.CompilerParams(vmem_limit_bytes=120 * 1024 * 1024)

T = 2048
C = 1024
N_HEAD = 16
D_PAD = 128
HP = N_HEAD * D_PAD
K_KEEP = 512
QB = 256
ROPE_HALF = 32
NOPE = 32
SCALE = 1.0 / (96.0 ** 0.5)
NEG = -1e9


def _pack_qk_weights(w_nope, w_rope):
    d = w_nope.shape[0]
    nope = w_nope.reshape(d, N_HEAD, NOPE)
    rope = w_rope.reshape(d, N_HEAD, 2 * ROPE_HALF)
    real = rope[:, :, :ROPE_HALF]
    imag = rope[:, :, ROPE_HALF:]
    z = jnp.zeros_like(nope)
    a = jnp.concatenate([nope, real, imag, z], axis=-1).reshape(d, HP)
    b = jnp.concatenate([z, imag, real, z], axis=-1).reshape(d, HP)
    return a, b


def _pack_qk_weights_fused(w):
    d = w.shape[0]
    w3 = w.reshape(d, N_HEAD, NOPE + 2 * ROPE_HALF)
    return _pack_qk_weights(
        w3[:, :, :NOPE].reshape(d, N_HEAD * NOPE),
        w3[:, :, NOPE:].reshape(d, N_HEAD * 2 * ROPE_HALF),
    )


def _pack_v_weights(w):
    d = w.shape[0]
    w3 = w.reshape(d, N_HEAD, 96)
    z = jnp.zeros((d, N_HEAD, D_PAD - 96), w.dtype)
    return jnp.concatenate([w3, z], axis=-1).reshape(d, HP)


def _rope_tables(n):
    freqs = 1.0 / 10000.0 ** (jnp.arange(0, 64, 2, dtype=jnp.float32) / 64.0)
    t = jnp.arange(n, dtype=jnp.float32)
    ang = jnp.outer(t, freqs)
    cos, sin = jnp.cos(ang), jnp.sin(ang)
    one = jnp.ones_like(cos)
    zero = jnp.zeros_like(cos)
    cos_blk = jnp.concatenate([one, cos, cos, zero], axis=-1)
    sin_blk = jnp.concatenate([zero, -sin, sin, zero], axis=-1)
    return jnp.tile(cos_blk, (1, N_HEAD)), jnp.tile(sin_blk, (1, N_HEAD))


def _proj_kernel(x_ref, wcq_ref, wqa_ref, wqb_ref, wckv_ref, wka_ref, wkb_ref,
                 wv_ref, wwa_ref, wwb_ref, wwv_ref, wimp_ref, wgate_ref,
                 cos_ref, sin_ref,
                 q_ref, k1_ref, v1_ref, kw_ref, vw_ref, imp_ref, gate_ref):
    bf16 = jnp.bfloat16
    xb = x_ref[...]
    xb16 = xb.astype(bf16)
    cosb = cos_ref[...]
    sinb = sin_ref[...]

    def rms(v):
        return jax.lax.rsqrt(jnp.mean(v * v, axis=-1, keepdims=True) + 1e-6)

    cq = jnp.dot(xb16, wcq_ref[...], preferred_element_type=jnp.float32)
    nq = (cq * rms(cq)).astype(bf16)
    q_ref[...] = ((jnp.dot(nq, wqa_ref[...], preferred_element_type=jnp.float32) * cosb
                   + jnp.dot(nq, wqb_ref[...], preferred_element_type=jnp.float32) * sinb)
                  ).astype(bf16)

    ckv = jnp.dot(xb16, wckv_ref[...], preferred_element_type=jnp.float32)
    nkv = (ckv * rms(ckv)).astype(bf16)
    k1_ref[...] = ((jnp.dot(nkv, wka_ref[...], preferred_element_type=jnp.float32) * cosb
                    + jnp.dot(nkv, wkb_ref[...], preferred_element_type=jnp.float32) * sinb)
                   ).astype(bf16)
    v1_ref[...] = jnp.dot(nkv, wv_ref[...], preferred_element_type=jnp.float32).astype(bf16)

    kw_ref[...] = ((jnp.dot(xb16, wwa_ref[...], preferred_element_type=jnp.float32) * cosb
                    + jnp.dot(xb16, wwb_ref[...], preferred_element_type=jnp.float32) * sinb)
                   ).astype(bf16)
    vw_ref[...] = jnp.dot(xb16, wwv_ref[...], preferred_element_type=jnp.float32).astype(bf16)

    imp_ref[...] = jnp.dot(xb, wimp_ref[...], preferred_element_type=jnp.float32)
    gp = jnp.dot(xb, wgate_ref[...], preferred_element_type=jnp.float32)
    gate_ref[...] = jnp.sum(gp, axis=0, keepdims=True).reshape(1, 1, 128)


def _topk_kernel(icol_ref, irow_ref, x_ref, cnt_ref, selx_ref):
    fcol = icol_ref[:, :1]
    frow = irow_ref[:1, :]
    isub = jax.lax.broadcasted_iota(jnp.int32, (T, T), 0)
    jlane = jax.lax.broadcasted_iota(jnp.int32, (T, T), 1)
    beats = jnp.where(
        (frow > fcol) | ((frow == fcol) & (jlane < isub)), 1.0, 0.0)
    rank_col = jnp.sum(beats, axis=1, keepdims=True)
    rank_row = (T - 1.0) - jnp.sum(beats, axis=0, keepdims=True)
    sel_col = jnp.where(rank_col < K_KEEP, 1.0, 0.0)
    sel_row = jnp.where(rank_row < K_KEEP, 1.0, 0.0)

    m = jnp.where(isub < jlane, 1.0, 0.0)
    sel_col128 = jnp.broadcast_to(sel_col, (T, 128))
    after = jnp.dot(m, sel_col128, preferred_element_type=jnp.float32)
    cnt_ref[...] = K_KEEP - after

    sel_row8 = jnp.broadcast_to(sel_row, (8, T))
    order8 = jnp.dot(sel_row8, m, preferred_element_type=jnp.float32)
    riota = jax.lax.broadcasted_iota(jnp.int32, (K_KEEP, T), 0).astype(jnp.float32)
    onehot = jnp.where((order8[:1, :] == riota) & (sel_row[:1, :] > 0.5), 1.0, 0.0)
    selx_ref[...] = jnp.dot(onehot, x_ref[...], preferred_element_type=jnp.float32)


def _selproj_kernel(selx_ref, wsa_ref, wsb_ref, wsv_ref, cos_ref, sin_ref,
                    ks_ref, vs_ref):
    sx = selx_ref[...].astype(jnp.bfloat16)
    ks_ref[...] = ((jnp.dot(sx, wsa_ref[...], preferred_element_type=jnp.float32) * cos_ref[...]
                    + jnp.dot(sx, wsb_ref[...], preferred_element_type=jnp.float32) * sin_ref[...])
                   ).astype(jnp.bfloat16)
    vs_ref[...] = jnp.dot(sx, wsv_ref[...], preferred_element_type=jnp.float32).astype(jnp.bfloat16)


def _attn_kernel(q_ref, k1_ref, v1_ref, kw_ref, vw_ref, ks_ref, vs_ref,
                 cnt_ref, bw_ref, o_ref):
    qb = pl.program_id(1)
    qv = q_ref[...]
    row = qb * QB + jax.lax.broadcasted_iota(jnp.int32, (QB, T), 0)
    col = jax.lax.broadcasted_iota(jnp.int32, (QB, T), 1)
    causal = col <= row

    dims = (((1,), (1,)), ((), ()))

    def soft_attend(k, v, mask):
        s = jax.lax.dot_general(qv, k, dims, preferred_element_type=jnp.float32) * SCALE
        s = jnp.where(mask, s, NEG)
        m = jnp.max(s, axis=-1, keepdims=True)
        p = jnp.exp(s - m)
        p = (p / jnp.sum(p, axis=-1, keepdims=True)).astype(jnp.bfloat16)
        return jnp.dot(p, v, preferred_element_type=jnp.float32)

    o1 = soft_attend(k1_ref[...], v1_ref[...], causal)
    o3 = soft_attend(kw_ref[...], vw_ref[...], causal)

    cnt = cnt_ref[:, :1]
    kidx = jax.lax.broadcasted_iota(jnp.int32, (QB, K_KEEP), 1).astype(jnp.float32)
    o2 = soft_attend(ks_ref[...], vs_ref[...], kidx < cnt)

    w1 = bw_ref[:1, 0:128]
    w2 = bw_ref[:1, 128:256]
    w3 = bw_ref[:1, 256:384]
    o_ref[...] = (o1 * w1 + o2 * w2 + o3 * w3).astype(jnp.bfloat16)


def _outproj_kernel(o_ref, wp_ref, out_ref):
    out_ref[...] = jnp.dot(o_ref[...], wp_ref[...],
                           preferred_element_type=jnp.float32)


@functools.partial(jax.jit, static_argnames=())
def kernel(x, W_cq, s_q, W_dq_nope, W_dq_rope, W_ckv, s_kv, W_dk_nope, W_dv,
           W_krope, W_imp, b_imp, W_selk, W_selv, W_wink, W_winv, W_gate,
           b_gate, W_proj):
    f32 = jnp.float32
    x2 = x.reshape(T, C).astype(f32)

    wqa, wqb = _pack_qk_weights(W_dq_nope, W_dq_rope)
    wqa, wqb = s_q[:, None] * wqa, s_q[:, None] * wqb
    wka, wkb = _pack_qk_weights(W_dk_nope, W_krope)
    wka, wkb = s_kv[:, None] * wka, s_kv[:, None] * wkb
    wv = s_kv[:, None] * _pack_v_weights(W_dv)
    wwa, wwb = _pack_qk_weights_fused(W_wink)
    wwv = _pack_v_weights(W_winv)
    wsa, wsb = _pack_qk_weights_fused(W_selk)
    wsv = _pack_v_weights(W_selv)
    wp = jnp.concatenate(
        [W_proj.reshape(N_HEAD, 96, C),
         jnp.zeros((N_HEAD, D_PAD - 96, C), f32)], axis=1).reshape(HP, C)
    wimp = jnp.concatenate([W_imp, jnp.zeros((C, 127), f32)], axis=-1)
    wgate = jnp.concatenate([W_gate, jnp.zeros((C, 125), f32)], axis=-1)
    cos_t, sin_t = _rope_tables(T)

    bf16 = jnp.bfloat16
    W_cq16, wqa, wqb = W_cq.astype(bf16), wqa.astype(bf16), wqb.astype(bf16)
    W_ckv16, wka, wkb = W_ckv.astype(bf16), wka.astype(bf16), wkb.astype(bf16)
    wv, wwa, wwb, wwv = (w.astype(bf16) for w in (wv, wwa, wwb, wwv))
    wsa, wsb, wsv, wp = (w.astype(bf16) for w in (wsa, wsb, wsv, wp))

    nblk = T // QB

    def full2(shape):
        return pl.BlockSpec(shape, lambda i: (0, 0))

    q, k1, v1, kw, vw, imp, gate_p = pl.pallas_call(
        _proj_kernel,
        compiler_params=_VMEM_BIG,
        grid=(nblk,),
        in_specs=[
            pl.BlockSpec((QB, C), lambda i: (i, 0)),
            full2((C, 96)), full2((96, HP)), full2((96, HP)),
            full2((C, 32)), full2((32, HP)), full2((32, HP)), full2((32, HP)),
            full2((C, HP)), full2((C, HP)), full2((C, HP)),
            full2((C, 128)), full2((C, 128)),
            pl.BlockSpec((QB, HP), lambda i: (i, 0)),
            pl.BlockSpec((QB, HP), lambda i: (i, 0)),
        ],
        out_specs=[
            pl.BlockSpec((QB, HP), lambda i: (i, 0)),
            pl.BlockSpec((QB, HP), lambda i: (i, 0)),
            pl.BlockSpec((QB, HP), lambda i: (i, 0)),
            pl.BlockSpec((QB, HP), lambda i: (i, 0)),
            pl.BlockSpec((QB, HP), lambda i: (i, 0)),
            pl.BlockSpec((QB, 128), lambda i: (i, 0)),
            pl.BlockSpec((1, 1, 128), lambda i: (i, 0, 0)),
        ],
        out_shape=[
            jax.ShapeDtypeStruct((T, HP), bf16),
            jax.ShapeDtypeStruct((T, HP), bf16),
            jax.ShapeDtypeStruct((T, HP), bf16),
            jax.ShapeDtypeStruct((T, HP), bf16),
            jax.ShapeDtypeStruct((T, HP), bf16),
            jax.ShapeDtypeStruct((T, 128), f32),
            jax.ShapeDtypeStruct((nblk, 1, 128), f32),
        ],
    )(x2, W_cq16, wqa, wqb, W_ckv16, wka, wkb, wv, wwa, wwb, wwv, wimp, wgate,
      cos_t, sin_t)

    glog = gate_p.reshape(nblk, 128).sum(axis=0)[:3] / T + b_gate
    bw3 = jax.nn.softmax(glog)
    bw = jnp.broadcast_to(jnp.repeat(bw3, 128)[None, :], (8, 384))

    imp_vec = imp[:, 0] + b_imp[0]
    icol = jnp.broadcast_to(imp_vec[:, None], (T, 128))
    irow = jnp.broadcast_to(imp_vec[None, :], (8, T))

    cnt, selx = pl.pallas_call(
        _topk_kernel,
        compiler_params=_VMEM_BIG,
        grid=(1,),
        in_specs=[full2((T, 128)), full2((8, T)), full2((T, C))],
        out_specs=[full2((T, 128)), full2((K_KEEP, C))],
        out_shape=[
            jax.ShapeDtypeStruct((T, 128), f32),
            jax.ShapeDtypeStruct((K_KEEP, C), f32),
        ],
    )(icol, irow, x2)

    ks, vs = pl.pallas_call(
        _selproj_kernel,
        compiler_params=_VMEM_BIG,
        grid=(1,),
        in_specs=[full2((K_KEEP, C)), full2((C, HP)), full2((C, HP)),
                  full2((C, HP)), full2((K_KEEP, HP)), full2((K_KEEP, HP))],
        out_specs=[full2((K_KEEP, HP)), full2((K_KEEP, HP))],
        out_shape=[
            jax.ShapeDtypeStruct((K_KEEP, HP), bf16),
            jax.ShapeDtypeStruct((K_KEEP, HP), bf16),
        ],
    )(selx, wsa, wsb, wsv, cos_t[:K_KEEP], sin_t[:K_KEEP])

    o = pl.pallas_call(
        _attn_kernel,
        grid=(N_HEAD, nblk),
        in_specs=[
            pl.BlockSpec((QB, D_PAD), lambda h, i: (i, h)),
            pl.BlockSpec((T, D_PAD), lambda h, i: (0, h)),
            pl.BlockSpec((T, D_PAD), lambda h, i: (0, h)),
            pl.BlockSpec((T, D_PAD), lambda h, i: (0, h)),
            pl.BlockSpec((T, D_PAD), lambda h, i: (0, h)),
            pl.BlockSpec((K_KEEP, D_PAD), lambda h, i: (0, h)),
            pl.BlockSpec((K_KEEP, D_PAD), lambda h, i: (0, h)),
            pl.BlockSpec((QB, 128), lambda h, i: (i, 0)),
            pl.BlockSpec((8, 384), lambda h, i: (0, 0)),
        ],
        out_specs=pl.BlockSpec((QB, D_PAD), lambda h, i: (i, h)),
        out_shape=jax.ShapeDtypeStruct((T, HP), bf16),
    )(q, k1, v1, kw, vw, ks, vs, cnt, bw)

    out = pl.pallas_call(
        _outproj_kernel,
        grid=(nblk,),
        in_specs=[pl.BlockSpec((QB, HP), lambda i: (i, 0)), full2((HP, C))],
        out_specs=pl.BlockSpec((QB, C), lambda i: (i, 0)),
        out_shape=jax.ShapeDtypeStruct((T, C), f32),
    )(o, wp)

    return out.reshape(1, T, C)
